# baseline (speedup 1.0000x reference)
import math

import jax
import jax.numpy as jnp
from jax import lax
from jax.experimental import pallas as pl
from jax.experimental.pallas import tpu as pltpu
from jax.experimental.pallas import tpu_sc as plsc

N = 10000
D = 128
EPS = 1e-5

NS = 16
CHUNK = 128
NCHUNK = 160
EDGES_PER_TILE = CHUNK * NCHUNK
E_PAD = NS * EDGES_PER_TILE
NPAD = 10240
ROWS_PER_TILE = NPAD // NS
_BN_INV = 1.0 / math.sqrt(1.0 + EPS)


def _sc_body(table, srcr, dstr, sum_out,
             src_v, dst_v, rows_v, sum_sh, gsem, ssem, isem):
    s = lax.axis_index("s")
    zero16 = jnp.zeros((16,), jnp.float32)
    base = s * ROWS_PER_TILE
    cbase = s * NCHUNK

    def zero_row(r, _):
        for k in range(D // 16):
            rows_v[r, pl.ds(k * 16, 16)] = zero16
        return 0

    lax.fori_loop(0, CHUNK, zero_row, 0)
    for k in range(ROWS_PER_TILE // CHUNK):
        pltpu.sync_copy(rows_v, sum_sh.at[pl.ds(base + k * CHUNK, CHUNK)])
    plsc.subcore_barrier()

    def chunk_body(j, _):
        ci = pltpu.async_copy(srcr.at[cbase + j], src_v.at[0], isem)
        pltpu.async_copy(dstr.at[cbase + j], dst_v.at[0], isem).wait()
        ci.wait()
        pltpu.async_copy(table.at[src_v.at[0]], rows_v, gsem).wait()
        pltpu.sync_copy(rows_v, sum_sh.at[dst_v.at[0]], add=True)
        return 0

    lax.fori_loop(0, NCHUNK, chunk_body, 0)
    plsc.subcore_barrier()

    for k in range(ROWS_PER_TILE // CHUNK):
        pltpu.sync_copy(sum_sh.at[pl.ds(base + k * CHUNK, CHUNK)], rows_v)
        pltpu.sync_copy(rows_v, sum_out.at[pl.ds(base + k * CHUNK, CHUNK)])


def _make_sc_pass():
    mesh = plsc.VectorSubcoreMesh(
        core_axis_name="c", subcore_axis_name="s", num_cores=1)
    out_type = [jax.ShapeDtypeStruct((NPAD, D), jnp.float32)]
    scratch = [
        pltpu.VMEM((2, CHUNK), jnp.int32),
        pltpu.VMEM((2, CHUNK), jnp.int32),
        pltpu.VMEM((CHUNK, D), jnp.float32),
        pltpu.VMEM_SHARED((NPAD, D), jnp.float32),
        pltpu.SemaphoreType.DMA,
        pltpu.SemaphoreType.DMA,
        pltpu.SemaphoreType.DMA,
    ]
    return pl.kernel(_sc_body, mesh=mesh, out_type=out_type,
                     scratch_types=scratch)


_sc_pass = _make_sc_pass()

R = 1000
GRID = N // R


def _tc1_body(sum_ref, cnt_ref, x_ref, wl_ref, wr_ref, wres_ref,
              g_ref, b_ref, bres_ref, hrelu_ref, h1_ref):
    cnt = cnt_ref[:, 0:1]
    agg = sum_ref[...] / jnp.maximum(cnt, 1.0)
    x = x_ref[...]
    h = (jnp.dot(agg, wl_ref[...], preferred_element_type=jnp.float32)
         + jnp.dot(x, wr_ref[...], preferred_element_type=jnp.float32))
    h = g_ref[...] * (h * _BN_INV) + b_ref[...]
    hr = jnp.maximum(h, 0.0)
    hrelu_ref[...] = hr
    h1_ref[...] = hr + jnp.dot(x, wres_ref[...],
                               preferred_element_type=jnp.float32) + bres_ref[...]


def _tc2_body(sum_ref, cnt_ref, x_ref, h1_ref, hr1_ref, wl_ref, wr_ref,
              g2_ref, b2_ref, wm0a_ref, wm0b_ref, wm0c_ref, bm0_ref,
              gm_ref, bm_ref, wm1_ref, bm1_ref, out_ref):
    cnt = cnt_ref[:, 0:1]
    agg = sum_ref[...] / jnp.maximum(cnt, 1.0)
    h1 = h1_ref[...]
    h2 = (jnp.dot(agg, wl_ref[...], preferred_element_type=jnp.float32)
          + jnp.dot(h1, wr_ref[...], preferred_element_type=jnp.float32))
    h2 = g2_ref[...] * (h2 * _BN_INV) + b2_ref[...]
    h2 = jnp.maximum(h2, 0.0)
    z1 = (jnp.dot(x_ref[...], wm0a_ref[...], preferred_element_type=jnp.float32)
          + jnp.dot(hr1_ref[...], wm0b_ref[...], preferred_element_type=jnp.float32)
          + jnp.dot(h2, wm0c_ref[...], preferred_element_type=jnp.float32)
          + bm0_ref[...])
    z1 = gm_ref[...] * (z1 * _BN_INV) + bm_ref[...]
    z1 = jnp.maximum(z1, 0.0)
    out_ref[...] = jnp.dot(z1, wm1_ref[...],
                           preferred_element_type=jnp.float32) + bm1_ref[...]


def _row_spec(shape):
    return pl.BlockSpec(shape, lambda i: (i, 0))


def _full_spec(shape):
    nd = len(shape)
    return pl.BlockSpec(shape, lambda i: (0,) * nd)


def _tc1(sums, cnts, x, Wl1, Wr1, Wres, g1, b1, bres):
    return pl.pallas_call(
        _tc1_body,
        grid=(GRID,),
        in_specs=[
            _row_spec((R, D)), _row_spec((R, D)), _row_spec((R, D)),
            _full_spec((D, D)), _full_spec((D, D)), _full_spec((D, D)),
            _full_spec((1, D)), _full_spec((1, D)), _full_spec((1, D)),
        ],
        out_specs=[_row_spec((R, D)), _row_spec((R, D))],
        out_shape=[jax.ShapeDtypeStruct((N, D), jnp.float32),
                   jax.ShapeDtypeStruct((N, D), jnp.float32)],
    )(sums, cnts, x, Wl1, Wr1, Wres, g1, b1, bres)


def _tc2(sums, cnts, x, h1, hr1, Wl2, Wr2, g2, b2,
         Wm0a, Wm0b, Wm0c, bm0, gm, bm, Wm1, bm1):
    cdim = Wm0a.shape[1]
    cout = Wm1.shape[1]
    return pl.pallas_call(
        _tc2_body,
        grid=(GRID,),
        in_specs=[
            _row_spec((R, D)), _row_spec((R, D)), _row_spec((R, D)),
            _row_spec((R, D)), _row_spec((R, D)),
            _full_spec((D, D)), _full_spec((D, D)),
            _full_spec((1, D)), _full_spec((1, D)),
            _full_spec((D, cdim)), _full_spec((D, cdim)), _full_spec((D, cdim)),
            _full_spec((1, cdim)), _full_spec((1, cdim)), _full_spec((1, cdim)),
            _full_spec((cdim, cout)), _full_spec((1, cout)),
        ],
        out_specs=[_row_spec((R, cout))],
        out_shape=[jax.ShapeDtypeStruct((N, cout), jnp.float32)],
    )(sums, cnts, x, h1, hr1, Wl2, Wr2, g2, b2,
      Wm0a, Wm0b, Wm0c, bm0, gm, bm, Wm1, bm1)[0]


def kernel(x, edge_index, Wl1, Wr1, g1, b1, Wl2, Wr2, g2, b2,
           Wres, bres, Wm0, bm0, gm, bm, Wm1, bm1):
    E = edge_index.shape[1]
    src = jnp.concatenate(
        [edge_index[0], jnp.zeros((E_PAD - E,), jnp.int32)])
    dst = jnp.concatenate(
        [edge_index[1], jnp.full((E_PAD - E,), N, jnp.int32)])
    srcr = src.reshape(NS * NCHUNK, CHUNK)
    dstr = dst.reshape(NS * NCHUNK, CHUNK)
    ones_tab = jnp.ones((N, D), jnp.float32)
    srcz = jnp.zeros_like(srcr)
    srcr, dstr, ones_tab, srcz = lax.optimization_barrier(
        (srcr, dstr, ones_tab, srcz))

    cnts = _sc_pass(ones_tab, srcz, dstr)[0][:N]
    sums1 = _sc_pass(x, srcr, dstr)[0][:N]

    hr1, h1 = _tc1(sums1, cnts, x, Wl1, Wr1, Wres,
                   g1.reshape(1, D), b1.reshape(1, D), bres.reshape(1, D))

    sums2 = _sc_pass(h1, srcr, dstr)[0][:N]

    cdim = Wm0.shape[1]
    cout = Wm1.shape[1]
    return _tc2(sums2, cnts, x, h1, hr1, Wl2, Wr2,
                g2.reshape(1, D), b2.reshape(1, D),
                Wm0[:D], Wm0[D:2 * D], Wm0[2 * D:], bm0.reshape(1, cdim),
                gm.reshape(1, cdim), bm.reshape(1, cdim),
                Wm1, bm1.reshape(1, cout))


# speedup vs baseline: 6.1442x; 6.1442x over previous
import math

import jax
import jax.numpy as jnp
from jax import lax
from jax.experimental import pallas as pl
from jax.experimental.pallas import tpu as pltpu
from jax.experimental.pallas import tpu_sc as plsc

N = 10000
D = 128
EPS = 1e-5

NS = 16
CHUNK = 128
NCHUNK = 160
EDGES_PER_TILE = CHUNK * NCHUNK
E_PAD = NS * EDGES_PER_TILE
NPAD = 10240
ROWS_PER_TILE = NPAD // NS
_BN_INV = 1.0 / math.sqrt(1.0 + EPS)


def _sc_body(table, srcr, dstr, sum_out,
             src_v, dst_v, rows_v, sum_sh, gsem, ssem, isem):
    s = lax.axis_index("s")
    zero16 = jnp.zeros((16,), jnp.float32)
    base = s * ROWS_PER_TILE
    cbase = s * NCHUNK

    def zero_row(r, _):
        for k in range(D // 16):
            rows_v[r, pl.ds(k * 16, 16)] = zero16
        return 0

    lax.fori_loop(0, CHUNK, zero_row, 0)
    for k in range(ROWS_PER_TILE // CHUNK):
        pltpu.sync_copy(rows_v, sum_sh.at[pl.ds(base + k * CHUNK, CHUNK)])
    plsc.subcore_barrier()

    def chunk_body(j, _):
        ci = pltpu.async_copy(srcr.at[cbase + j], src_v.at[0], isem)
        pltpu.async_copy(dstr.at[cbase + j], dst_v.at[0], isem).wait()
        ci.wait()
        pltpu.async_copy(table.at[src_v.at[0]], rows_v, gsem).wait()
        pltpu.sync_copy(rows_v, sum_sh.at[dst_v.at[0]], add=True)
        return 0

    lax.fori_loop(0, NCHUNK, chunk_body, 0)
    plsc.subcore_barrier()

    for k in range(ROWS_PER_TILE // CHUNK):
        pltpu.sync_copy(sum_sh.at[pl.ds(base + k * CHUNK, CHUNK)], rows_v)
        pltpu.sync_copy(rows_v, sum_out.at[pl.ds(base + k * CHUNK, CHUNK)])


def _make_sc_pass():
    mesh = plsc.VectorSubcoreMesh(
        core_axis_name="c", subcore_axis_name="s", num_cores=1)
    out_type = [jax.ShapeDtypeStruct((NPAD, D), jnp.float32)]
    scratch = [
        pltpu.VMEM((2, CHUNK), jnp.int32),
        pltpu.VMEM((2, CHUNK), jnp.int32),
        pltpu.VMEM((CHUNK, D), jnp.float32),
        pltpu.VMEM_SHARED((NPAD, D), jnp.float32),
        pltpu.SemaphoreType.DMA,
        pltpu.SemaphoreType.DMA,
        pltpu.SemaphoreType.DMA,
    ]
    return pl.kernel(_sc_body, mesh=mesh, out_type=out_type,
                     scratch_types=scratch)


_sc_pass = _make_sc_pass()

R = 1000
GRID = N // R


def _tc1_body(sum_ref, cnt_ref, x_ref, wl_ref, wr_ref, wres_ref,
              g_ref, b_ref, bres_ref, hrelu_ref, h1_ref):
    cnt = cnt_ref[:, 0:1]
    agg = sum_ref[...] / jnp.maximum(cnt, 1.0)
    x = x_ref[...]
    h = (jnp.dot(agg, wl_ref[...], preferred_element_type=jnp.float32)
         + jnp.dot(x, wr_ref[...], preferred_element_type=jnp.float32))
    h = g_ref[...] * (h * _BN_INV) + b_ref[...]
    hr = jnp.maximum(h, 0.0)
    hrelu_ref[...] = hr
    h1_ref[...] = hr + jnp.dot(x, wres_ref[...],
                               preferred_element_type=jnp.float32) + bres_ref[...]


def _tc2_body(sum_ref, cnt_ref, x_ref, h1_ref, hr1_ref, wl_ref, wr_ref,
              g2_ref, b2_ref, wm0a_ref, wm0b_ref, wm0c_ref, bm0_ref,
              gm_ref, bm_ref, wm1_ref, bm1_ref, out_ref):
    cnt = cnt_ref[:, 0:1]
    agg = sum_ref[...] / jnp.maximum(cnt, 1.0)
    h1 = h1_ref[...]
    h2 = (jnp.dot(agg, wl_ref[...], preferred_element_type=jnp.float32)
          + jnp.dot(h1, wr_ref[...], preferred_element_type=jnp.float32))
    h2 = g2_ref[...] * (h2 * _BN_INV) + b2_ref[...]
    h2 = jnp.maximum(h2, 0.0)
    z1 = (jnp.dot(x_ref[...], wm0a_ref[...], preferred_element_type=jnp.float32)
          + jnp.dot(hr1_ref[...], wm0b_ref[...], preferred_element_type=jnp.float32)
          + jnp.dot(h2, wm0c_ref[...], preferred_element_type=jnp.float32)
          + bm0_ref[...])
    z1 = gm_ref[...] * (z1 * _BN_INV) + bm_ref[...]
    z1 = jnp.maximum(z1, 0.0)
    out_ref[...] = jnp.dot(z1, wm1_ref[...],
                           preferred_element_type=jnp.float32) + bm1_ref[...]


def _row_spec(shape):
    return pl.BlockSpec(shape, lambda i: (i, 0))


def _full_spec(shape):
    nd = len(shape)
    return pl.BlockSpec(shape, lambda i: (0,) * nd)


def _tc1(sums, cnts, x, Wl1, Wr1, Wres, g1, b1, bres):
    return pl.pallas_call(
        _tc1_body,
        grid=(GRID,),
        in_specs=[
            _row_spec((R, D)), _row_spec((R, D)), _row_spec((R, D)),
            _full_spec((D, D)), _full_spec((D, D)), _full_spec((D, D)),
            _full_spec((1, D)), _full_spec((1, D)), _full_spec((1, D)),
        ],
        out_specs=[_row_spec((R, D)), _row_spec((R, D))],
        out_shape=[jax.ShapeDtypeStruct((N, D), jnp.float32),
                   jax.ShapeDtypeStruct((N, D), jnp.float32)],
    )(sums, cnts, x, Wl1, Wr1, Wres, g1, b1, bres)


def _tc2(sums, cnts, x, h1, hr1, Wl2, Wr2, g2, b2,
         Wm0a, Wm0b, Wm0c, bm0, gm, bm, Wm1, bm1):
    cdim = Wm0a.shape[1]
    cout = Wm1.shape[1]
    return pl.pallas_call(
        _tc2_body,
        grid=(GRID,),
        in_specs=[
            _row_spec((R, D)), _row_spec((R, D)), _row_spec((R, D)),
            _row_spec((R, D)), _row_spec((R, D)),
            _full_spec((D, D)), _full_spec((D, D)),
            _full_spec((1, D)), _full_spec((1, D)),
            _full_spec((D, cdim)), _full_spec((D, cdim)), _full_spec((D, cdim)),
            _full_spec((1, cdim)), _full_spec((1, cdim)), _full_spec((1, cdim)),
            _full_spec((cdim, cout)), _full_spec((1, cout)),
        ],
        out_specs=[_row_spec((R, cout))],
        out_shape=[jax.ShapeDtypeStruct((N, cout), jnp.float32)],
    )(sums, cnts, x, h1, hr1, Wl2, Wr2, g2, b2,
      Wm0a, Wm0b, Wm0c, bm0, gm, bm, Wm1, bm1)[0]


def kernel(x, edge_index, Wl1, Wr1, g1, b1, Wl2, Wr2, g2, b2,
           Wres, bres, Wm0, bm0, gm, bm, Wm1, bm1):
    E = edge_index.shape[1]
    src = jnp.concatenate(
        [edge_index[0], jnp.zeros((E_PAD - E,), jnp.int32)])
    dst = jnp.concatenate(
        [edge_index[1], jnp.full((E_PAD - E,), N, jnp.int32)])
    srcr = src.reshape(NS * NCHUNK, CHUNK)
    dstr = dst.reshape(NS * NCHUNK, CHUNK)
    ones_tab = jnp.ones((N, D), jnp.float32)
    srcr, dstr, ones_tab = lax.optimization_barrier((srcr, dstr, ones_tab))

    cnts = _sc_pass(ones_tab, srcr, dstr)[0][:N]
    sums1 = _sc_pass(x, srcr, dstr)[0][:N]

    hr1, h1 = _tc1(sums1, cnts, x, Wl1, Wr1, Wres,
                   g1.reshape(1, D), b1.reshape(1, D), bres.reshape(1, D))

    sums2 = _sc_pass(h1, srcr, dstr)[0][:N]

    cdim = Wm0.shape[1]
    cout = Wm1.shape[1]
    return _tc2(sums2, cnts, x, h1, hr1, Wl2, Wr2,
                g2.reshape(1, D), b2.reshape(1, D),
                Wm0[:D], Wm0[D:2 * D], Wm0[2 * D:], bm0.reshape(1, cdim),
                gm.reshape(1, cdim), bm.reshape(1, cdim),
                Wm1, bm1.reshape(1, cout))


# speedup vs baseline: 7.3366x; 1.1941x over previous
import math

import jax
import jax.numpy as jnp
from jax import lax
from jax.experimental import pallas as pl
from jax.experimental.pallas import tpu as pltpu
from jax.experimental.pallas import tpu_sc as plsc

N = 10000
D = 128
EPS = 1e-5

NS = 16
CHUNK = 128
NCHUNK = 160
EDGES_PER_TILE = CHUNK * NCHUNK
E_PAD = NS * EDGES_PER_TILE
NPAD = 10240
ROWS_PER_TILE = NPAD // NS
_BN_INV = 1.0 / math.sqrt(1.0 + EPS)


def _sc_body(table, srcr, dstr, sum_out,
             src_v, dst_v, rows_a, rows_b, sum_sh, gsem, ssem, isem):
    s = lax.axis_index("s")
    zero16 = jnp.zeros((16,), jnp.float32)
    base = s * ROWS_PER_TILE
    cbase = s * NCHUNK

    def zero_row(r, _):
        for k in range(D // 16):
            rows_a[r, pl.ds(k * 16, 16)] = zero16
        return 0

    lax.fori_loop(0, CHUNK, zero_row, 0)
    for k in range(ROWS_PER_TILE // CHUNK):
        pltpu.sync_copy(rows_a, sum_sh.at[pl.ds(base + k * CHUNK, CHUNK)])
    plsc.subcore_barrier()

    def two(i, _):
        j0 = 2 * i
        c0 = pltpu.async_copy(srcr.at[cbase + j0], src_v.at[0], isem)
        c1 = pltpu.async_copy(dstr.at[cbase + j0], dst_v.at[0], isem)
        c2 = pltpu.async_copy(srcr.at[cbase + j0 + 1], src_v.at[1], isem)
        c3 = pltpu.async_copy(dstr.at[cbase + j0 + 1], dst_v.at[1], isem)
        c0.wait()
        g0 = pltpu.async_copy(table.at[src_v.at[0]], rows_a, gsem)
        c2.wait()
        g1 = pltpu.async_copy(table.at[src_v.at[1]], rows_b, gsem)
        c1.wait()
        c3.wait()
        g0.wait()
        s0 = pltpu.async_copy(rows_a, sum_sh.at[dst_v.at[0]], ssem, add=True)
        g1.wait()
        s1 = pltpu.async_copy(rows_b, sum_sh.at[dst_v.at[1]], ssem, add=True)
        s0.wait()
        s1.wait()
        return 0

    lax.fori_loop(0, NCHUNK // 2, two, 0)
    plsc.subcore_barrier()

    for k in range(ROWS_PER_TILE // CHUNK):
        pltpu.sync_copy(sum_sh.at[pl.ds(base + k * CHUNK, CHUNK)], rows_a)
        pltpu.sync_copy(rows_a, sum_out.at[pl.ds(base + k * CHUNK, CHUNK)])


def _make_sc_pass():
    mesh = plsc.VectorSubcoreMesh(
        core_axis_name="c", subcore_axis_name="s", num_cores=1)
    out_type = [jax.ShapeDtypeStruct((NPAD, D), jnp.float32)]
    scratch = [
        pltpu.VMEM((2, CHUNK), jnp.int32),
        pltpu.VMEM((2, CHUNK), jnp.int32),
        pltpu.VMEM((CHUNK, D), jnp.float32),
        pltpu.VMEM((CHUNK, D), jnp.float32),
        pltpu.VMEM_SHARED((NPAD, D), jnp.float32),
        pltpu.SemaphoreType.DMA,
        pltpu.SemaphoreType.DMA,
        pltpu.SemaphoreType.DMA,
    ]
    return pl.kernel(_sc_body, mesh=mesh, out_type=out_type,
                     scratch_types=scratch)


_sc_pass = _make_sc_pass()

R = 1000
GRID = N // R


def _tc1_body(sum_ref, cnt_ref, x_ref, wl_ref, wr_ref, wres_ref,
              g_ref, b_ref, bres_ref, hrelu_ref, h1_ref):
    cnt = cnt_ref[:, 0:1]
    agg = sum_ref[...] / jnp.maximum(cnt, 1.0)
    x = x_ref[...]
    h = (jnp.dot(agg, wl_ref[...], preferred_element_type=jnp.float32)
         + jnp.dot(x, wr_ref[...], preferred_element_type=jnp.float32))
    h = g_ref[...] * (h * _BN_INV) + b_ref[...]
    hr = jnp.maximum(h, 0.0)
    hrelu_ref[...] = hr
    h1_ref[...] = hr + jnp.dot(x, wres_ref[...],
                               preferred_element_type=jnp.float32) + bres_ref[...]


def _tc2_body(sum_ref, cnt_ref, x_ref, h1_ref, hr1_ref, wl_ref, wr_ref,
              g2_ref, b2_ref, wm0a_ref, wm0b_ref, wm0c_ref, bm0_ref,
              gm_ref, bm_ref, wm1_ref, bm1_ref, out_ref):
    cnt = cnt_ref[:, 0:1]
    agg = sum_ref[...] / jnp.maximum(cnt, 1.0)
    h1 = h1_ref[...]
    h2 = (jnp.dot(agg, wl_ref[...], preferred_element_type=jnp.float32)
          + jnp.dot(h1, wr_ref[...], preferred_element_type=jnp.float32))
    h2 = g2_ref[...] * (h2 * _BN_INV) + b2_ref[...]
    h2 = jnp.maximum(h2, 0.0)
    z1 = (jnp.dot(x_ref[...], wm0a_ref[...], preferred_element_type=jnp.float32)
          + jnp.dot(hr1_ref[...], wm0b_ref[...], preferred_element_type=jnp.float32)
          + jnp.dot(h2, wm0c_ref[...], preferred_element_type=jnp.float32)
          + bm0_ref[...])
    z1 = gm_ref[...] * (z1 * _BN_INV) + bm_ref[...]
    z1 = jnp.maximum(z1, 0.0)
    out_ref[...] = jnp.dot(z1, wm1_ref[...],
                           preferred_element_type=jnp.float32) + bm1_ref[...]


def _row_spec(shape):
    return pl.BlockSpec(shape, lambda i: (i, 0))


def _full_spec(shape):
    nd = len(shape)
    return pl.BlockSpec(shape, lambda i: (0,) * nd)


def _tc1(sums, cnts, x, Wl1, Wr1, Wres, g1, b1, bres):
    return pl.pallas_call(
        _tc1_body,
        grid=(GRID,),
        in_specs=[
            _row_spec((R, D)), _row_spec((R, D)), _row_spec((R, D)),
            _full_spec((D, D)), _full_spec((D, D)), _full_spec((D, D)),
            _full_spec((1, D)), _full_spec((1, D)), _full_spec((1, D)),
        ],
        out_specs=[_row_spec((R, D)), _row_spec((R, D))],
        out_shape=[jax.ShapeDtypeStruct((N, D), jnp.float32),
                   jax.ShapeDtypeStruct((N, D), jnp.float32)],
    )(sums, cnts, x, Wl1, Wr1, Wres, g1, b1, bres)


def _tc2(sums, cnts, x, h1, hr1, Wl2, Wr2, g2, b2,
         Wm0a, Wm0b, Wm0c, bm0, gm, bm, Wm1, bm1):
    cdim = Wm0a.shape[1]
    cout = Wm1.shape[1]
    return pl.pallas_call(
        _tc2_body,
        grid=(GRID,),
        in_specs=[
            _row_spec((R, D)), _row_spec((R, D)), _row_spec((R, D)),
            _row_spec((R, D)), _row_spec((R, D)),
            _full_spec((D, D)), _full_spec((D, D)),
            _full_spec((1, D)), _full_spec((1, D)),
            _full_spec((D, cdim)), _full_spec((D, cdim)), _full_spec((D, cdim)),
            _full_spec((1, cdim)), _full_spec((1, cdim)), _full_spec((1, cdim)),
            _full_spec((cdim, cout)), _full_spec((1, cout)),
        ],
        out_specs=[_row_spec((R, cout))],
        out_shape=[jax.ShapeDtypeStruct((N, cout), jnp.float32)],
    )(sums, cnts, x, h1, hr1, Wl2, Wr2, g2, b2,
      Wm0a, Wm0b, Wm0c, bm0, gm, bm, Wm1, bm1)[0]


def kernel(x, edge_index, Wl1, Wr1, g1, b1, Wl2, Wr2, g2, b2,
           Wres, bres, Wm0, bm0, gm, bm, Wm1, bm1):
    E = edge_index.shape[1]
    src = jnp.concatenate(
        [edge_index[0], jnp.zeros((E_PAD - E,), jnp.int32)])
    dst = jnp.concatenate(
        [edge_index[1], jnp.full((E_PAD - E,), N, jnp.int32)])
    srcr = src.reshape(NS * NCHUNK, CHUNK)
    dstr = dst.reshape(NS * NCHUNK, CHUNK)
    ones_tab = jnp.ones((N, D), jnp.float32)
    srcr, dstr, ones_tab = lax.optimization_barrier((srcr, dstr, ones_tab))

    cnts = _sc_pass(ones_tab, srcr, dstr)[0][:N]
    sums1 = _sc_pass(x, srcr, dstr)[0][:N]

    hr1, h1 = _tc1(sums1, cnts, x, Wl1, Wr1, Wres,
                   g1.reshape(1, D), b1.reshape(1, D), bres.reshape(1, D))

    sums2 = _sc_pass(h1, srcr, dstr)[0][:N]

    cdim = Wm0.shape[1]
    cout = Wm1.shape[1]
    return _tc2(sums2, cnts, x, h1, hr1, Wl2, Wr2,
                g2.reshape(1, D), b2.reshape(1, D),
                Wm0[:D], Wm0[D:2 * D], Wm0[2 * D:], bm0.reshape(1, cdim),
                gm.reshape(1, cdim), bm.reshape(1, cdim),
                Wm1, bm1.reshape(1, cout))
